# baseline (speedup 1.0000x reference)
import functools

import jax
import jax.numpy as jnp
from jax import lax
from jax.experimental import pallas as pl
from jax.experimental.pallas import tpu as pltpu
from jax.experimental.pallas import tpu_sc as plsc

N_GRAPHS = 1024
NODES_PER_GRAPH = 128
D_FEAT = 128
LANES = 16
NC, NS = 2, 16
NW = NC * NS
GPW = N_GRAPHS // NW
CH = D_FEAT // LANES
SCALE = 1.0 / NODES_PER_GRAPH

GPD = 2
PAIRS = GPW // GPD
ROWS = NODES_PER_GRAPH * GPD
NBUF = 2


def _pool_body(feats_hbm, nb_hbm, out_hbm, idx_v, rows_a, rows_b, out_v,
               sem_a, sem_b):
    wid = lax.axis_index("s") * NC + lax.axis_index("c")
    base = wid * PAIRS
    pltpu.sync_copy(nb_hbm.at[pl.ds(base, PAIRS)], idx_v)
    bufs = (rows_a, rows_b)
    sems = (sem_a, sem_b)
    def start(j, buf, sem):
        for h in range(GPD):
            pltpu.async_copy(
                feats_hbm.at[idx_v.at[j, pl.ds(h * NODES_PER_GRAPH,
                                               NODES_PER_GRAPH)]],
                buf.at[pl.ds(h * NODES_PER_GRAPH, NODES_PER_GRAPH)], sem)

    for b in range(NBUF):
        start(b, bufs[b], sems[b])

    def super_body(t, carry):
        for b in range(NBUF):
            j = t * NBUF + b
            rows_v = bufs[b]
            pltpu.make_async_copy(
                feats_hbm.at[pl.ds(0, ROWS)], rows_v, sems[b]).wait()
            for h in range(GPD):

                def body(r, accs):
                    return tuple(
                        accs[c] + rows_v[r, pl.ds(c * LANES, LANES)]
                        for c in range(CH))

                accs = lax.fori_loop(
                    h * NODES_PER_GRAPH, (h + 1) * NODES_PER_GRAPH, body,
                    tuple(jnp.zeros((LANES,), jnp.float32)
                          for _ in range(CH)))
                for c in range(CH):
                    out_v[j * GPD + h, pl.ds(c * LANES, LANES)] = (
                        accs[c] * SCALE)
            jn = j + NBUF

            @pl.when(jn < PAIRS)
            def _():
                start(jn, rows_v, sems[b])

        return carry

    lax.fori_loop(0, PAIRS // NBUF, super_body, 0)
    pltpu.sync_copy(out_v, out_hbm.at[pl.ds(wid * GPW, GPW)])


@jax.jit
def kernel(feats, node_batches):
    nb2 = node_batches.reshape(N_GRAPHS // GPD, NODES_PER_GRAPH * GPD)
    mesh = plsc.VectorSubcoreMesh(core_axis_name="c", subcore_axis_name="s")
    f = pl.kernel(
        _pool_body,
        mesh=mesh,
        out_type=jax.ShapeDtypeStruct((N_GRAPHS, D_FEAT), jnp.float32),
        scratch_types=[
            pltpu.VMEM((PAIRS, ROWS), jnp.int32),
        ] + [pltpu.VMEM((ROWS, D_FEAT), jnp.float32)] * NBUF + [
            pltpu.VMEM((GPW, D_FEAT), jnp.float32),
        ] + [pltpu.SemaphoreType.DMA] * NBUF,
    )
    return f(feats, nb2)


# speedup vs baseline: 1.1033x; 1.1033x over previous
import jax
import jax.numpy as jnp
from jax import lax
from jax.experimental import pallas as pl
from jax.experimental.pallas import tpu as pltpu
from jax.experimental.pallas import tpu_sc as plsc

N_GRAPHS = 1024
NODES_PER_GRAPH = 128
D_FEAT = 128
LANES = 16
NC, NS = 2, 16
NW = NC * NS
GPW = N_GRAPHS // NW
CH = D_FEAT // LANES
SCALE = 1.0 / NODES_PER_GRAPH
NBUF = 4


def _pool_body(feats_hbm, nb_hbm, out_hbm, idx_v, rows_a, rows_b, rows_c,
               rows_d, out_v, sem_a, sem_b, sem_c, sem_d):
    wid = lax.axis_index("s") * NC + lax.axis_index("c")
    base = wid * GPW
    pltpu.sync_copy(nb_hbm.at[pl.ds(base, GPW)], idx_v)
    bufs = (rows_a, rows_b, rows_c, rows_d)
    sems = (sem_a, sem_b, sem_c, sem_d)
    for b in range(NBUF):
        pltpu.async_copy(feats_hbm.at[idx_v.at[b]], bufs[b], sems[b])

    def super_body(t, carry):
        for b in range(NBUF):
            g = t * NBUF + b
            rows_v = bufs[b]
            pltpu.make_async_copy(
                feats_hbm.at[pl.ds(0, NODES_PER_GRAPH)], rows_v,
                sems[b]).wait()

            def body(r, accs):
                return tuple(accs[c] + rows_v[r, pl.ds(c * LANES, LANES)]
                             for c in range(CH))

            accs = lax.fori_loop(
                0, NODES_PER_GRAPH, body,
                tuple(jnp.zeros((LANES,), jnp.float32) for _ in range(CH)))
            for c in range(CH):
                out_v[g, pl.ds(c * LANES, LANES)] = accs[c] * SCALE
            gn = g + NBUF

            @pl.when(gn < GPW)
            def _():
                pltpu.async_copy(feats_hbm.at[idx_v.at[gn]], rows_v, sems[b])

        return carry

    lax.fori_loop(0, GPW // NBUF, super_body, 0)
    pltpu.sync_copy(out_v, out_hbm.at[pl.ds(base, GPW)])


@jax.jit
def kernel(feats, node_batches):
    mesh = plsc.VectorSubcoreMesh(core_axis_name="c", subcore_axis_name="s")
    f = pl.kernel(
        _pool_body,
        mesh=mesh,
        out_type=jax.ShapeDtypeStruct((N_GRAPHS, D_FEAT), jnp.float32),
        scratch_types=[
            pltpu.VMEM((GPW, NODES_PER_GRAPH), jnp.int32),
        ] + [pltpu.VMEM((NODES_PER_GRAPH, D_FEAT), jnp.float32)] * NBUF + [
            pltpu.VMEM((GPW, D_FEAT), jnp.float32),
        ] + [pltpu.SemaphoreType.DMA] * NBUF,
    )
    return f(feats, node_batches)


# speedup vs baseline: 1.1067x; 1.0031x over previous
import jax
import jax.numpy as jnp
from jax import lax
from jax.experimental import pallas as pl
from jax.experimental.pallas import tpu as pltpu
from jax.experimental.pallas import tpu_sc as plsc

N_GRAPHS = 1024
NODES_PER_GRAPH = 128
D_FEAT = 128
LANES = 16
NC, NS = 2, 16
NW = NC * NS
GPW = N_GRAPHS // NW
CH = D_FEAT // LANES
SCALE = 1.0 / NODES_PER_GRAPH
NBUF = 4


def _pool_body(feats_hbm, nb_hbm, out_hbm, idx_v, rows_a, rows_b, rows_c,
               rows_d, out_v, sem_a, sem_b, sem_c, sem_d):
    wid = lax.axis_index("s") * NC + lax.axis_index("c")
    base = wid * GPW
    pltpu.sync_copy(nb_hbm.at[pl.ds(base, GPW)], idx_v)
    bufs = (rows_a, rows_b, rows_c, rows_d)
    sems = (sem_a, sem_b, sem_c, sem_d)
    for b in range(NBUF):
        pltpu.async_copy(feats_hbm.at[idx_v.at[b]], bufs[b], sems[b])

    def super_body(t, carry):
        for b in range(NBUF):
            g = t * NBUF + b
            rows_v = bufs[b]
            pltpu.make_async_copy(
                feats_hbm.at[pl.ds(0, NODES_PER_GRAPH)], rows_v,
                sems[b]).wait()

            def body(r, accs):
                return tuple(accs[c] + rows_v[r, pl.ds(c * LANES, LANES)]
                             for c in range(CH))

            accs = lax.fori_loop(
                0, NODES_PER_GRAPH, body,
                tuple(jnp.zeros((LANES,), jnp.float32) for _ in range(CH)))
            gn = g + NBUF

            @pl.when(gn < GPW)
            def _():
                pltpu.async_copy(feats_hbm.at[idx_v.at[gn]], rows_v, sems[b])

            for c in range(CH):
                out_v[g, pl.ds(c * LANES, LANES)] = accs[c] * SCALE

        return carry

    lax.fori_loop(0, GPW // NBUF, super_body, 0)
    pltpu.sync_copy(out_v, out_hbm.at[pl.ds(base, GPW)])


@jax.jit
def kernel(feats, node_batches):
    mesh = plsc.VectorSubcoreMesh(core_axis_name="c", subcore_axis_name="s")
    f = pl.kernel(
        _pool_body,
        mesh=mesh,
        out_type=jax.ShapeDtypeStruct((N_GRAPHS, D_FEAT), jnp.float32),
        scratch_types=[
            pltpu.VMEM((GPW, NODES_PER_GRAPH), jnp.int32),
        ] + [pltpu.VMEM((NODES_PER_GRAPH, D_FEAT), jnp.float32)] * NBUF + [
            pltpu.VMEM((GPW, D_FEAT), jnp.float32),
        ] + [pltpu.SemaphoreType.DMA] * NBUF,
    )
    return f(feats, node_batches)
